# baseline (speedup 1.0000x reference)
import functools
import math

import jax
import jax.numpy as jnp
from jax import lax
from jax.experimental import pallas as pl
from jax.experimental.pallas import tpu as pltpu
from jax.experimental.pallas import tpu_sc as plsc

_NUM_CORES = 2
_NUM_SUBCORES = 16
_NUM_WORKERS = _NUM_CORES * _NUM_SUBCORES
_LANES = 16
_CHUNK = 128
_NBUF = 5
_PD = 3


def _on_stream(j):
  return j % 2 == 0


def _make_sc_lookup(B, L, V, D):
  mesh = plsc.VectorSubcoreMesh(core_axis_name="c", subcore_axis_name="s")
  t_per_w = B * L // _NUM_WORKERS
  n_chunks = t_per_w // _CHUNK
  period = L // math.gcd(_CHUNK, L)
  assert t_per_w % _CHUNK == 0 and n_chunks % period == 0
  assert period % _NBUF == 0 and _PD < _NBUF

  @functools.partial(
      pl.kernel,
      out_type=jax.ShapeDtypeStruct((B * L, D), jnp.float32),
      mesh=mesh,
      scratch_types=[
          pltpu.VMEM_SHARED((L, D), jnp.float32),
          pltpu.VMEM((L, D), jnp.float32),
          pltpu.VMEM((t_per_w,), jnp.int32),
      ] + [pltpu.VMEM((_CHUNK, D), jnp.float32) for _ in range(_NBUF)]
        + [pltpu.SemaphoreType.DMA for _ in range(2 * _NBUF)],
  )
  def lookup(x_hbm, pe_hbm, emb_hbm, out_hbm, pe_sh, pe_v, idx_v, *bufs_sems):
    rows = bufs_sems[:_NBUF]
    gsem = bufs_sems[_NBUF:2 * _NBUF]
    ssem = bufs_sems[2 * _NBUF:]
    wid = lax.axis_index("s") * _NUM_CORES + lax.axis_index("c")
    base = wid * t_per_w
    pltpu.sync_copy(x_hbm.at[pl.ds(base, t_per_w)], idx_v)
    pltpu.sync_copy(pe_hbm, pe_v)

    @pl.when(lax.axis_index("s") == 0)
    def _():
      pltpu.sync_copy(pe_hbm, pe_sh)
    plsc.subcore_barrier()

    def pe_segments(j):
      off = (j * _CHUNK) % L
      sz1 = min(_CHUNK, L - off)
      segs = [(0, off, sz1)]
      if sz1 < _CHUNK:
        segs.append((sz1, 0, _CHUNK - sz1))
      return segs

    def gather_start(c, j):
      p = j % _NBUF
      jp = j % period
      if _on_stream(jp):
        for dst_lo, src_lo, n in pe_segments(jp):
          pltpu.sync_copy(pe_sh.at[pl.ds(src_lo, n)],
                          rows[p].at[pl.ds(dst_lo, n)])
      pltpu.async_copy(
          emb_hbm.at[idx_v.at[pl.ds(c * _CHUNK, _CHUNK)]],
          rows[p], gsem[p], add=_on_stream(jp))

    def add_pe(j):
      p = j % _NBUF
      buf = rows[p]

      def add_rows(dst_lo, src_lo, n):
        def row(r, _):
          for d in range(D // _LANES):
            sl = pl.ds(d * _LANES, _LANES)
            buf[dst_lo + r, sl] = buf[dst_lo + r, sl] + pe_v[src_lo + r, sl]
          return 0
        lax.fori_loop(0, n, row, 0)

      for dst_lo, src_lo, n in pe_segments(j):
        add_rows(dst_lo, src_lo, n)

    def gather_drain(p):
      pltpu.make_async_copy(
          emb_hbm.at[pl.ds(0, _CHUNK)], rows[p], gsem[p]).wait()

    def store_start(c, p):
      pltpu.async_copy(
          rows[p], out_hbm.at[pl.ds(base + c * _CHUNK, _CHUNK)], ssem[p])

    def store_drain(p):
      pltpu.make_async_copy(
          emb_hbm.at[pl.ds(0, _CHUNK)], rows[p], ssem[p]).wait()

    for c in range(_PD):
      gather_start(c, c)

    @pl.loop(0, n_chunks, step=period)
    def _(c0):
      for j in range(period):
        c = c0 + j
        p = j % _NBUF
        q = (j + _PD) % _NBUF

        @pl.when(c >= _NBUF - _PD)
        def _():
          store_drain(q)

        @pl.when(c + _PD < n_chunks)
        def _():
          gather_start(c + _PD, j + _PD)

        gather_drain(p)
        if not _on_stream(j):
          add_pe(j)
        store_start(c, p)

    for i in range(_NBUF - _PD):
      store_drain((n_chunks - (_NBUF - _PD) + i) % _NBUF)

  return lookup


def kernel(x, offset, emb, pe):
  B, L = x.shape
  V, D = emb.shape
  pe_s = lax.dynamic_slice_in_dim(pe, offset, L, axis=0)
  out = _make_sc_lookup(B, L, V, D)(x.reshape(-1), pe_s, emb)
  return out.reshape(B, L, D)


# speedup vs baseline: 1.1060x; 1.1060x over previous
import functools
import math

import jax
import jax.numpy as jnp
from jax import lax
from jax.experimental import pallas as pl
from jax.experimental.pallas import tpu as pltpu
from jax.experimental.pallas import tpu_sc as plsc

_NUM_CORES = 2
_NUM_SUBCORES = 16
_NUM_WORKERS = _NUM_CORES * _NUM_SUBCORES
_CHUNK = 128
_NBUF = 5
_PDP = 3
_PDG = 2


def _make_sc_lookup(B, L, V, D):
  mesh = plsc.VectorSubcoreMesh(core_axis_name="c", subcore_axis_name="s")
  t_per_w = B * L // _NUM_WORKERS
  n_chunks = t_per_w // _CHUNK
  period = L // math.gcd(_CHUNK, L)
  assert t_per_w % _CHUNK == 0 and n_chunks % period == 0
  assert period % _NBUF == 0 and _PDG < _PDP < _NBUF - 1

  @functools.partial(
      pl.kernel,
      out_type=jax.ShapeDtypeStruct((B * L, D), jnp.float32),
      mesh=mesh,
      scratch_types=[
          pltpu.VMEM_SHARED((L, D), jnp.float32),
          pltpu.VMEM((t_per_w,), jnp.int32),
      ] + [pltpu.VMEM((_CHUNK, D), jnp.float32) for _ in range(_NBUF)]
        + [pltpu.SemaphoreType.DMA for _ in range(3 * _NBUF)],
  )
  def lookup(x_hbm, pe_hbm, emb_hbm, out_hbm, pe_sh, idx_v, *bufs_sems):
    rows = bufs_sems[:_NBUF]
    psem = bufs_sems[_NBUF:2 * _NBUF]
    gsem = bufs_sems[2 * _NBUF:3 * _NBUF]
    ssem = bufs_sems[3 * _NBUF:]
    wid = lax.axis_index("s") * _NUM_CORES + lax.axis_index("c")
    base = wid * t_per_w
    pltpu.sync_copy(x_hbm.at[pl.ds(base, t_per_w)], idx_v)

    @pl.when(lax.axis_index("s") == 0)
    def _():
      pltpu.sync_copy(pe_hbm, pe_sh)
    plsc.subcore_barrier()

    def prefill_start(j):
      p = j % _NBUF
      jp = j % period
      off = (jp * _CHUNK) % L
      sz1 = min(_CHUNK, L - off)
      pltpu.async_copy(pe_sh.at[pl.ds(off, sz1)],
                       rows[p].at[pl.ds(0, sz1)], psem[p])
      if sz1 < _CHUNK:
        pltpu.async_copy(pe_sh.at[pl.ds(0, _CHUNK - sz1)],
                         rows[p].at[pl.ds(sz1, _CHUNK - sz1)], psem[p])

    def prefill_wait(p):
      pltpu.make_async_copy(
          emb_hbm.at[pl.ds(0, _CHUNK)], rows[p], psem[p]).wait()

    def gather_start(c, j):
      p = j % _NBUF
      pltpu.async_copy(
          emb_hbm.at[idx_v.at[pl.ds(c * _CHUNK, _CHUNK)]],
          rows[p], gsem[p], add=True)

    def gather_drain(p):
      pltpu.make_async_copy(
          emb_hbm.at[pl.ds(0, _CHUNK)], rows[p], gsem[p]).wait()

    def store_start(c, p):
      pltpu.async_copy(
          rows[p], out_hbm.at[pl.ds(base + c * _CHUNK, _CHUNK)], ssem[p])

    def store_drain(p):
      pltpu.make_async_copy(
          emb_hbm.at[pl.ds(0, _CHUNK)], rows[p], ssem[p]).wait()

    for m in range(_PDP):
      prefill_start(m)
    for m in range(_PDG):
      prefill_wait(m % _NBUF)
      gather_start(m, m)

    @pl.loop(0, n_chunks, step=period)
    def _(c0):
      for j in range(period):
        c = c0 + j

        @pl.when(c + _PDP < n_chunks)
        def _():
          @pl.when(c >= _NBUF - _PDP)
          def _():
            store_drain((j + _PDP) % _NBUF)
          prefill_start(j + _PDP)

        @pl.when(c + _PDG < n_chunks)
        def _():
          prefill_wait((j + _PDG) % _NBUF)
          gather_start(c + _PDG, j + _PDG)

        gather_drain(j % _NBUF)
        store_start(c, j % _NBUF)

    for i in range(_NBUF):
      store_drain((n_chunks - _NBUF + i) % _NBUF)

  return lookup


def kernel(x, offset, emb, pe):
  B, L = x.shape
  V, D = emb.shape
  pe_s = lax.dynamic_slice_in_dim(pe, offset, L, axis=0)
  out = _make_sc_lookup(B, L, V, D)(x.reshape(-1), pe_s, emb)
  return out.reshape(B, L, D)


# speedup vs baseline: 1.1273x; 1.0193x over previous
import functools

import jax
import jax.numpy as jnp
from jax import lax
from jax.experimental import pallas as pl
from jax.experimental.pallas import tpu as pltpu
from jax.experimental.pallas import tpu_sc as plsc

_NUM_CORES = 2
_NUM_SUBCORES = 16
_NUM_WORKERS = _NUM_CORES * _NUM_SUBCORES
_NBUF = 8
_PD = 4


def _make_sc_lookup(B, L, V, D):
  mesh = plsc.VectorSubcoreMesh(core_axis_name="c", subcore_axis_name="s")
  b_per_w = B // _NUM_WORKERS
  ch0 = ((L // 2) + 7) // 8 * 8
  ch1 = L - ch0
  assert 0 < ch1 <= 128 and ch0 <= 128 and ch0 % 8 == 0
  n_chunks = 2 * b_per_w
  assert n_chunks % _NBUF == 0 and _PD % 2 == 0 and _PD < _NBUF
  chunk_sz = (ch0, ch1)
  chunk_off = (0, ch0)

  @functools.partial(
      pl.kernel,
      out_type=jax.ShapeDtypeStruct((B, L, D), jnp.float32),
      mesh=mesh,
      scratch_types=[
          pltpu.VMEM_SHARED((L, D), jnp.float32),
          pltpu.VMEM((b_per_w * L,), jnp.int32),
      ] + [pltpu.VMEM((ch0, D), jnp.float32) for _ in range(_NBUF)]
        + [pltpu.SemaphoreType.DMA for _ in range(2 * _NBUF)],
  )
  def lookup(x_hbm, pe_hbm, emb_hbm, out_hbm, pe_sh, idx_v, *bufs_sems):
    rows = bufs_sems[:_NBUF]
    gsem = bufs_sems[_NBUF:2 * _NBUF]
    ssem = bufs_sems[2 * _NBUF:]
    wid = lax.axis_index("s") * _NUM_CORES + lax.axis_index("c")
    base = wid * b_per_w
    pltpu.sync_copy(x_hbm.at[pl.ds(base * L, b_per_w * L)], idx_v)

    @pl.when(lax.axis_index("s") == 0)
    def _():
      pltpu.sync_copy(pe_hbm, pe_sh)
    plsc.subcore_barrier()

    def gather_start(nb, p):
      sz = chunk_sz[p & 1]
      pltpu.sync_copy(pe_sh.at[pl.ds(chunk_off[p & 1], sz)],
                      rows[p].at[pl.ds(0, sz)])
      pltpu.async_copy(
          emb_hbm.at[idx_v.at[pl.ds(nb * L + chunk_off[p & 1], sz)]],
          rows[p].at[pl.ds(0, sz)], gsem[p], add=True)

    def gather_drain(p):
      sz = chunk_sz[p & 1]
      pltpu.make_async_copy(
          emb_hbm.at[pl.ds(0, sz)], rows[p].at[pl.ds(0, sz)], gsem[p]).wait()

    def store_start(nb, p):
      sz = chunk_sz[p & 1]
      pltpu.async_copy(
          rows[p].at[pl.ds(0, sz)],
          out_hbm.at[base + nb, pl.ds(chunk_off[p & 1], sz)], ssem[p])

    def store_drain(p):
      sz = chunk_sz[p & 1]
      pltpu.make_async_copy(
          emb_hbm.at[pl.ds(0, sz)], rows[p].at[pl.ds(0, sz)], ssem[p]).wait()

    for c in range(_PD):
      gather_start(c // 2, c)

    @pl.loop(0, n_chunks, step=_NBUF)
    def _(c0):
      for j in range(_NBUF):
        c = c0 + j
        nb = c0 // 2 + (j // 2)
        gather_drain(j)
        store_start(nb, j)
        q = (j + _PD) % _NBUF
        nb_pre = c0 // 2 + (j + _PD) // 2

        @pl.when(c >= _NBUF - _PD)
        def _():
          store_drain(q)

        @pl.when(c + _PD < n_chunks)
        def _():
          gather_start(nb_pre, q)

    for p in range(_PD, _NBUF):
      store_drain(p)

  return lookup


def kernel(x, offset, emb, pe):
  B, L = x.shape
  V, D = emb.shape
  pe_s = lax.dynamic_slice_in_dim(pe, offset, L, axis=0)
  return _make_sc_lookup(B, L, V, D)(x.reshape(-1), pe_s, emb)
